# baseline (speedup 1.0000x reference)
import functools

import jax
import jax.numpy as jnp
from jax import lax
from jax.experimental import pallas as pl
from jax.experimental.pallas import tpu as pltpu
from jax.experimental.pallas import tpu_sc as plsc

N = 10000
H = 128
E = 320000

NPAD = 10112
NW = 32
CH = 128
NCH = 80
TPT = NCH * CH
EP = NW * TPT
RPT = NPAD // 16
NDEG = 10240
RPTD = NDEG // 16
BR = 632
GRID = NPAD // BR

_mesh = plsc.VectorSubcoreMesh(core_axis_name="c", subcore_axis_name="s",
                               num_cores=2, num_subcores=16)


@functools.partial(
    pl.kernel,
    out_type=[jax.ShapeDtypeStruct((NDEG,), jnp.float32),
              jax.ShapeDtypeStruct((NDEG,), jnp.float32)],
    mesh=_mesh,
    scratch_types=[
        pltpu.VMEM((NCH, CH), jnp.int32),
        pltpu.VMEM((CH,), jnp.float32),
        pltpu.VMEM_SHARED((NDEG,), jnp.float32),
        pltpu.SemaphoreType.DMA,
    ],
)
def _deg_kernel(dstm, ones_hbm, zeros_hbm, out0, out1, idx_d, ones_v, acc,
                sem):
    cid = lax.axis_index("c")
    sid = lax.axis_index("s")
    wid = cid * 16 + sid
    pltpu.sync_copy(zeros_hbm, acc.at[pl.ds(sid * RPTD, RPTD)])
    pltpu.sync_copy(ones_hbm, ones_v)
    pltpu.sync_copy(dstm.at[pl.ds(wid * NCH, NCH)], idx_d)
    plsc.subcore_barrier()

    def body(i, carry):
        pltpu.async_copy(ones_v, acc.at[idx_d.at[i]], sem, add=True)
        return carry

    lax.fori_loop(0, NCH, body, 0)

    def drain(i, carry):
        pltpu.make_async_copy(ones_v, acc.at[idx_d.at[i]], sem).wait()
        return carry

    lax.fori_loop(0, NCH, drain, 0)
    plsc.subcore_barrier()

    @pl.when(cid == 0)
    def _():
        pltpu.sync_copy(acc.at[pl.ds(sid * RPTD, RPTD)],
                        out0.at[pl.ds(sid * RPTD, RPTD)])

    @pl.when(cid == 1)
    def _():
        pltpu.sync_copy(acc.at[pl.ds(sid * RPTD, RPTD)],
                        out1.at[pl.ds(sid * RPTD, RPTD)])


@functools.partial(
    pl.kernel,
    out_type=jax.ShapeDtypeStruct((2, NPAD, H), jnp.float32),
    mesh=_mesh,
    scratch_types=[
        pltpu.VMEM((NCH, CH), jnp.int32),
        pltpu.VMEM((CH,), jnp.int32),
        pltpu.VMEM((CH,), jnp.int32),
        pltpu.VMEM((CH, H), jnp.float32),
        pltpu.VMEM((CH, H), jnp.float32),
        pltpu.VMEM_SHARED((NPAD, H), jnp.float32),
        pltpu.SemaphoreType.DMA,
        pltpu.SemaphoreType.DMA,
        pltpu.SemaphoreType.DMA,
        pltpu.SemaphoreType.DMA,
        pltpu.SemaphoreType.DMA,
        pltpu.SemaphoreType.DMA,
    ],
)
def _agg_kernel(h1s, srcm, dst1, zeros_hbm, out, idx_s, idx_d0, idx_d1,
                rows0, rows1, acc, sem0, sem1, semi0, semi1, sems0, sems1):
    cid = lax.axis_index("c")
    sid = lax.axis_index("s")
    wid = cid * 16 + sid
    base = wid * TPT
    @pl.when(cid == 0)
    def _():
        pltpu.sync_copy(h1s.at[pl.ds(sid * RPT, RPT)],
                        acc.at[pl.ds(sid * RPT, RPT)])

    @pl.when(cid == 1)
    def _():
        pltpu.sync_copy(zeros_hbm, acc.at[pl.ds(sid * RPT, RPT)])
    pltpu.sync_copy(srcm.at[pl.ds(wid * NCH, NCH)], idx_s)
    plsc.subcore_barrier()

    pltpu.async_copy(h1s.at[idx_s.at[0]], rows0, sem0)
    pltpu.async_copy(dst1.at[pl.ds(base, CH)], idx_d0, semi0)

    def body(j, carry):
        i0 = 2 * j
        pltpu.async_copy(h1s.at[idx_s.at[i0 + 1]], rows1, sem1)
        pltpu.async_copy(dst1.at[pl.ds(base + (i0 + 1) * CH, CH)], idx_d1,
                         semi1)
        pltpu.make_async_copy(h1s.at[idx_s.at[i0]], rows0, sem0).wait()
        pltpu.make_async_copy(dst1.at[pl.ds(base, CH)], idx_d0, semi0).wait()
        pltpu.async_copy(rows0, acc.at[idx_d0], sems0, add=True)
        pltpu.make_async_copy(h1s.at[idx_s.at[i0 + 1]], rows1, sem1).wait()
        pltpu.make_async_copy(dst1.at[pl.ds(base, CH)], idx_d1, semi1).wait()
        pltpu.async_copy(rows1, acc.at[idx_d1], sems1, add=True)
        pltpu.make_async_copy(rows0, acc.at[idx_d0], sems0).wait()

        @pl.when(i0 + 2 < NCH)
        def _():
            pltpu.async_copy(h1s.at[idx_s.at[i0 + 2]], rows0, sem0)
            pltpu.async_copy(dst1.at[pl.ds(base + (i0 + 2) * CH, CH)], idx_d0,
                             semi0)

        pltpu.make_async_copy(rows1, acc.at[idx_d1], sems1).wait()
        return carry

    lax.fori_loop(0, NCH // 2, body, 0)
    plsc.subcore_barrier()
    pltpu.sync_copy(acc.at[pl.ds(sid * RPT, RPT)],
                    out.at[cid, pl.ds(sid * RPT, RPT)])


def _prep_body(x_ref, w0_ref, degb0_ref, degb1_ref, t_ref, wg1_ref, bg1_ref,
               wg2_ref, bg2_ref, h1s_ref, dinvb_ref, gate_ref):
    pid = pl.program_id(0)
    deg = degb0_ref[...] + degb1_ref[...] + 1.0
    rows = jax.lax.broadcasted_iota(jnp.int32, (BR, H), 0) + pid * BR
    dinv = jnp.where(rows < N, jax.lax.rsqrt(deg), 0.0)
    mm = jax.lax.dot_general(x_ref[...], w0_ref[...], (((1,), (1,)), ((), ())),
                             preferred_element_type=jnp.float32)
    h1s_ref[...] = jnp.where(rows < N, dinv * mm, 0.0)
    dinvb_ref[...] = dinv
    t = t_ref[0, 0]
    g = jnp.tanh(t * wg1_ref[...] + bg1_ref[...])
    gate_ref[...] = jax.nn.sigmoid(
        jax.lax.dot_general(g, wg2_ref[...], (((1,), (1,)), ((), ())),
                            preferred_element_type=jnp.float32) + bg2_ref[...])


_prep_call = pl.pallas_call(
    _prep_body,
    grid=(GRID,),
    in_specs=[
        pl.BlockSpec((BR, H), lambda i: (i, 0)),
        pl.BlockSpec((H, H), lambda i: (0, 0)),
        pl.BlockSpec((BR, H), lambda i: (i, 0)),
        pl.BlockSpec((BR, H), lambda i: (i, 0)),
        pl.BlockSpec((1, 1), lambda i: (0, 0)),
        pl.BlockSpec((1, H), lambda i: (0, 0)),
        pl.BlockSpec((1, H), lambda i: (0, 0)),
        pl.BlockSpec((H, H), lambda i: (0, 0)),
        pl.BlockSpec((1, H), lambda i: (0, 0)),
    ],
    out_specs=[
        pl.BlockSpec((BR, H), lambda i: (i, 0)),
        pl.BlockSpec((BR, H), lambda i: (i, 0)),
        pl.BlockSpec((1, H), lambda i: (0, 0)),
    ],
    out_shape=[
        jax.ShapeDtypeStruct((NPAD, H), jnp.float32),
        jax.ShapeDtypeStruct((NPAD, H), jnp.float32),
        jax.ShapeDtypeStruct((1, H), jnp.float32),
    ],
)


def _layer_body(agg_ref, dinvb_ref, gate_ref, b_ref, w_ref, out_ref):
    s = agg_ref[0] + agg_ref[1]
    h = gate_ref[...] * jnp.maximum(dinvb_ref[...] * s + b_ref[...], 0.0)
    out_ref[...] = dinvb_ref[...] * jax.lax.dot_general(
        h, w_ref[...], (((1,), (1,)), ((), ())),
        preferred_element_type=jnp.float32)


_layer_call = pl.pallas_call(
    _layer_body,
    grid=(GRID,),
    in_specs=[
        pl.BlockSpec((2, BR, H), lambda i: (0, i, 0)),
        pl.BlockSpec((BR, H), lambda i: (i, 0)),
        pl.BlockSpec((1, H), lambda i: (0, 0)),
        pl.BlockSpec((1, H), lambda i: (0, 0)),
        pl.BlockSpec((H, H), lambda i: (0, 0)),
    ],
    out_specs=pl.BlockSpec((BR, H), lambda i: (i, 0)),
    out_shape=jax.ShapeDtypeStruct((NPAD, H), jnp.float32),
)


def _final_body(agg_ref, dinvb_ref, gate_ref, b_ref, out_ref):
    s = agg_ref[0] + agg_ref[1]
    out_ref[...] = gate_ref[...] * jnp.maximum(
        dinvb_ref[...] * s + b_ref[...], 0.0)


BRF = 1000

_final_call = pl.pallas_call(
    _final_body,
    grid=(N // BRF,),
    in_specs=[
        pl.BlockSpec((2, BRF, H), lambda i: (0, i, 0)),
        pl.BlockSpec((BRF, H), lambda i: (i, 0)),
        pl.BlockSpec((1, H), lambda i: (0, 0)),
        pl.BlockSpec((1, H), lambda i: (0, 0)),
    ],
    out_specs=pl.BlockSpec((BRF, H), lambda i: (i, 0)),
    out_shape=jax.ShapeDtypeStruct((N, H), jnp.float32),
)


def kernel(x, edge_index, timestamp, W0, b0, W1, b1, W2, b2, Wg1, bg1, Wg2, bg2):
    src = edge_index[0]
    dst = edge_index[1]
    pad_idx = N + (jnp.arange(EP - E, dtype=jnp.int32) % (NPAD - N))
    srcm = jnp.concatenate([src, pad_idx]).reshape(NW * NCH, CH)
    dst1 = jnp.concatenate([dst, pad_idx])
    dstm = dst1.reshape(NW * NCH, CH)
    zeros_h = jnp.zeros((RPT, H), jnp.float32)
    zeros_1 = jnp.zeros((RPTD,), jnp.float32)
    ones_1 = jnp.ones((CH,), jnp.float32)
    t = timestamp.reshape(1, 1)
    wg1r = Wg1.reshape(1, H)
    bg1r = bg1.reshape(1, H)
    bg2r = bg2.reshape(1, H)
    b0r = b0.reshape(1, H)
    b1r = b1.reshape(1, H)
    b2r = b2.reshape(1, H)

    deg0, deg1 = _deg_kernel(dstm, ones_1, zeros_1)
    degb0 = jnp.broadcast_to(deg0[:NPAD, None], (NPAD, H))
    degb1 = jnp.broadcast_to(deg1[:NPAD, None], (NPAD, H))
    h1s0, dinvb, gate = _prep_call(x, W0, degb0, degb1, t, wg1r, bg1r,
                                   Wg2, bg2r)
    agg0 = _agg_kernel(h1s0, srcm, dst1, zeros_h)
    h1s1 = _layer_call(agg0, dinvb, gate, b0r, W1)
    agg1 = _agg_kernel(h1s1, srcm, dst1, zeros_h)
    h1s2 = _layer_call(agg1, dinvb, gate, b1r, W2)
    agg2 = _agg_kernel(h1s2, srcm, dst1, zeros_h)
    return _final_call(agg2, dinvb, gate, b2r)


# speedup vs baseline: 1.3043x; 1.3043x over previous
import functools

import jax
import jax.numpy as jnp
from jax import lax
from jax.experimental import pallas as pl
from jax.experimental.pallas import tpu as pltpu
from jax.experimental.pallas import tpu_sc as plsc

N = 10000
H = 128
E = 320000

NPAD = 10112
NW = 32
CH = 128
NCH = 80
TPT = NCH * CH
EP = NW * TPT
RPT = NPAD // 16
NDEG = 10240
RPTD = NDEG // 16
BR = 1264
GRID = NPAD // BR

_mesh = plsc.VectorSubcoreMesh(core_axis_name="c", subcore_axis_name="s",
                               num_cores=2, num_subcores=16)


@functools.partial(
    pl.kernel,
    out_type=[jax.ShapeDtypeStruct((NDEG,), jnp.float32),
              jax.ShapeDtypeStruct((NDEG,), jnp.float32)],
    mesh=_mesh,
    scratch_types=[
        pltpu.VMEM((NCH, CH), jnp.int32),
        pltpu.VMEM((CH,), jnp.float32),
        pltpu.VMEM_SHARED((NDEG,), jnp.float32),
        pltpu.SemaphoreType.DMA,
    ],
)
def _deg_kernel(dstm, ones_hbm, zeros_hbm, out0, out1, idx_d, ones_v, acc,
                sem):
    cid = lax.axis_index("c")
    sid = lax.axis_index("s")
    wid = cid * 16 + sid
    pltpu.sync_copy(zeros_hbm, acc.at[pl.ds(sid * RPTD, RPTD)])
    pltpu.sync_copy(ones_hbm, ones_v)
    pltpu.sync_copy(dstm.at[pl.ds(wid * NCH, NCH)], idx_d)
    plsc.subcore_barrier()

    def body(i, carry):
        pltpu.async_copy(ones_v, acc.at[idx_d.at[i]], sem, add=True)
        return carry

    lax.fori_loop(0, NCH, body, 0)

    def drain(i, carry):
        pltpu.make_async_copy(ones_v, acc.at[idx_d.at[i]], sem).wait()
        return carry

    lax.fori_loop(0, NCH, drain, 0)
    plsc.subcore_barrier()

    @pl.when(cid == 0)
    def _():
        pltpu.sync_copy(acc.at[pl.ds(sid * RPTD, RPTD)],
                        out0.at[pl.ds(sid * RPTD, RPTD)])

    @pl.when(cid == 1)
    def _():
        pltpu.sync_copy(acc.at[pl.ds(sid * RPTD, RPTD)],
                        out1.at[pl.ds(sid * RPTD, RPTD)])


@functools.partial(
    pl.kernel,
    out_type=jax.ShapeDtypeStruct((2, NPAD, H), jnp.float32),
    mesh=_mesh,
    scratch_types=[
        pltpu.VMEM((NCH, CH), jnp.int32),
        pltpu.VMEM((CH,), jnp.int32),
        pltpu.VMEM((CH,), jnp.int32),
        pltpu.VMEM((CH, H), jnp.float32),
        pltpu.VMEM((CH, H), jnp.float32),
        pltpu.VMEM_SHARED((NPAD, H), jnp.float32),
        pltpu.SemaphoreType.DMA,
        pltpu.SemaphoreType.DMA,
        pltpu.SemaphoreType.DMA,
        pltpu.SemaphoreType.DMA,
    ],
)
def _agg_kernel(h1s, srcm, dst1, zeros_hbm, out, idx_s, idx_d0, idx_d1,
                rows0, rows1, acc, sem0, sem1, semi0, semi1):
    cid = lax.axis_index("c")
    sid = lax.axis_index("s")
    wid = cid * 16 + sid
    base = wid * TPT
    @pl.when(cid == 0)
    def _():
        pltpu.sync_copy(h1s.at[pl.ds(sid * RPT, RPT)],
                        acc.at[pl.ds(sid * RPT, RPT)])

    @pl.when(cid == 1)
    def _():
        pltpu.sync_copy(zeros_hbm, acc.at[pl.ds(sid * RPT, RPT)])
    pltpu.sync_copy(srcm.at[pl.ds(wid * NCH, NCH)], idx_s)
    plsc.subcore_barrier()

    pltpu.async_copy(h1s.at[idx_s.at[0]], rows0, sem0)
    pltpu.async_copy(dst1.at[pl.ds(base, CH)], idx_d0, semi0)

    def body(j, carry):
        i0 = 2 * j
        pltpu.async_copy(h1s.at[idx_s.at[i0 + 1]], rows1, sem1)
        pltpu.async_copy(dst1.at[pl.ds(base + (i0 + 1) * CH, CH)], idx_d1,
                         semi1)
        pltpu.make_async_copy(h1s.at[idx_s.at[i0]], rows0, sem0).wait()
        pltpu.make_async_copy(dst1.at[pl.ds(base, CH)], idx_d0, semi0).wait()
        pltpu.sync_copy(rows0, acc.at[idx_d0], add=True)

        @pl.when(i0 + 2 < NCH)
        def _():
            pltpu.async_copy(h1s.at[idx_s.at[i0 + 2]], rows0, sem0)
            pltpu.async_copy(dst1.at[pl.ds(base + (i0 + 2) * CH, CH)], idx_d0,
                             semi0)

        pltpu.make_async_copy(h1s.at[idx_s.at[i0 + 1]], rows1, sem1).wait()
        pltpu.make_async_copy(dst1.at[pl.ds(base, CH)], idx_d1, semi1).wait()
        pltpu.sync_copy(rows1, acc.at[idx_d1], add=True)
        return carry

    lax.fori_loop(0, NCH // 2, body, 0)
    plsc.subcore_barrier()
    pltpu.sync_copy(acc.at[pl.ds(sid * RPT, RPT)],
                    out.at[cid, pl.ds(sid * RPT, RPT)])


def _prep_body(x_ref, w0_ref, degb0_ref, degb1_ref, t_ref, wg1_ref, bg1_ref,
               wg2_ref, bg2_ref, h1s_ref, dinvb_ref, gate_ref):
    pid = pl.program_id(0)
    deg = degb0_ref[...] + degb1_ref[...] + 1.0
    rows = jax.lax.broadcasted_iota(jnp.int32, (BR, H), 0) + pid * BR
    dinv = jnp.where(rows < N, jax.lax.rsqrt(deg), 0.0)
    mm = jax.lax.dot_general(x_ref[...], w0_ref[...], (((1,), (1,)), ((), ())),
                             preferred_element_type=jnp.float32)
    h1s_ref[...] = jnp.where(rows < N, dinv * mm, 0.0)
    dinvb_ref[...] = dinv
    t = t_ref[0, 0]
    g = jnp.tanh(t * wg1_ref[...] + bg1_ref[...])
    gate_ref[...] = jax.nn.sigmoid(
        jax.lax.dot_general(g, wg2_ref[...], (((1,), (1,)), ((), ())),
                            preferred_element_type=jnp.float32) + bg2_ref[...])


_prep_call = pl.pallas_call(
    _prep_body,
    grid=(GRID,),
    in_specs=[
        pl.BlockSpec((BR, H), lambda i: (i, 0)),
        pl.BlockSpec((H, H), lambda i: (0, 0)),
        pl.BlockSpec((BR, H), lambda i: (i, 0)),
        pl.BlockSpec((BR, H), lambda i: (i, 0)),
        pl.BlockSpec((1, 1), lambda i: (0, 0)),
        pl.BlockSpec((1, H), lambda i: (0, 0)),
        pl.BlockSpec((1, H), lambda i: (0, 0)),
        pl.BlockSpec((H, H), lambda i: (0, 0)),
        pl.BlockSpec((1, H), lambda i: (0, 0)),
    ],
    out_specs=[
        pl.BlockSpec((BR, H), lambda i: (i, 0)),
        pl.BlockSpec((BR, H), lambda i: (i, 0)),
        pl.BlockSpec((1, H), lambda i: (0, 0)),
    ],
    out_shape=[
        jax.ShapeDtypeStruct((NPAD, H), jnp.float32),
        jax.ShapeDtypeStruct((NPAD, H), jnp.float32),
        jax.ShapeDtypeStruct((1, H), jnp.float32),
    ],
)


def _layer_body(agg_ref, dinvb_ref, gate_ref, b_ref, w_ref, out_ref):
    s = agg_ref[0] + agg_ref[1]
    h = gate_ref[...] * jnp.maximum(dinvb_ref[...] * s + b_ref[...], 0.0)
    out_ref[...] = dinvb_ref[...] * jax.lax.dot_general(
        h, w_ref[...], (((1,), (1,)), ((), ())),
        preferred_element_type=jnp.float32)


_layer_call = pl.pallas_call(
    _layer_body,
    grid=(GRID,),
    in_specs=[
        pl.BlockSpec((2, BR, H), lambda i: (0, i, 0)),
        pl.BlockSpec((BR, H), lambda i: (i, 0)),
        pl.BlockSpec((1, H), lambda i: (0, 0)),
        pl.BlockSpec((1, H), lambda i: (0, 0)),
        pl.BlockSpec((H, H), lambda i: (0, 0)),
    ],
    out_specs=pl.BlockSpec((BR, H), lambda i: (i, 0)),
    out_shape=jax.ShapeDtypeStruct((NPAD, H), jnp.float32),
)


def _final_body(agg_ref, dinvb_ref, gate_ref, b_ref, out_ref):
    s = agg_ref[0] + agg_ref[1]
    out_ref[...] = gate_ref[...] * jnp.maximum(
        dinvb_ref[...] * s + b_ref[...], 0.0)


BRF = 1000

_final_call = pl.pallas_call(
    _final_body,
    grid=(N // BRF,),
    in_specs=[
        pl.BlockSpec((2, BRF, H), lambda i: (0, i, 0)),
        pl.BlockSpec((BRF, H), lambda i: (i, 0)),
        pl.BlockSpec((1, H), lambda i: (0, 0)),
        pl.BlockSpec((1, H), lambda i: (0, 0)),
    ],
    out_specs=pl.BlockSpec((BRF, H), lambda i: (i, 0)),
    out_shape=jax.ShapeDtypeStruct((N, H), jnp.float32),
)


def kernel(x, edge_index, timestamp, W0, b0, W1, b1, W2, b2, Wg1, bg1, Wg2, bg2):
    src = edge_index[0]
    dst = edge_index[1]
    pad_idx = N + (jnp.arange(EP - E, dtype=jnp.int32) % (NPAD - N))
    srcm = jnp.concatenate([src, pad_idx]).reshape(NW * NCH, CH)
    dst1 = jnp.concatenate([dst, pad_idx])
    dstm = dst1.reshape(NW * NCH, CH)
    zeros_h = jnp.zeros((RPT, H), jnp.float32)
    zeros_1 = jnp.zeros((RPTD,), jnp.float32)
    ones_1 = jnp.ones((CH,), jnp.float32)
    t = timestamp.reshape(1, 1)
    wg1r = Wg1.reshape(1, H)
    bg1r = bg1.reshape(1, H)
    bg2r = bg2.reshape(1, H)
    b0r = b0.reshape(1, H)
    b1r = b1.reshape(1, H)
    b2r = b2.reshape(1, H)

    deg0, deg1 = _deg_kernel(dstm, ones_1, zeros_1)
    degb0 = jnp.broadcast_to(deg0[:NPAD, None], (NPAD, H))
    degb1 = jnp.broadcast_to(deg1[:NPAD, None], (NPAD, H))
    h1s0, dinvb, gate = _prep_call(x, W0, degb0, degb1, t, wg1r, bg1r,
                                   Wg2, bg2r)
    agg0 = _agg_kernel(h1s0, srcm, dst1, zeros_h)
    h1s1 = _layer_call(agg0, dinvb, gate, b0r, W1)
    agg1 = _agg_kernel(h1s1, srcm, dst1, zeros_h)
    h1s2 = _layer_call(agg1, dinvb, gate, b1r, W2)
    agg2 = _agg_kernel(h1s2, srcm, dst1, zeros_h)
    return _final_call(agg2, dinvb, gate, b2r)


# speedup vs baseline: 1.3161x; 1.0091x over previous
import functools

import jax
import jax.numpy as jnp
from jax import lax
from jax.experimental import pallas as pl
from jax.experimental.pallas import tpu as pltpu
from jax.experimental.pallas import tpu_sc as plsc

N = 10000
H = 128
E = 320000

NPAD = 10112
NW = 32
CH = 128
NCH = 80
TPT = NCH * CH
EP = NW * TPT
RPT = NPAD // 16
NDEG = 10240
RPTD = NDEG // 16
BR = 2528
GRID = NPAD // BR

_mesh = plsc.VectorSubcoreMesh(core_axis_name="c", subcore_axis_name="s",
                               num_cores=2, num_subcores=16)


@functools.partial(
    pl.kernel,
    out_type=[jax.ShapeDtypeStruct((NDEG,), jnp.float32),
              jax.ShapeDtypeStruct((NDEG,), jnp.float32)],
    mesh=_mesh,
    scratch_types=[
        pltpu.VMEM((NCH, CH), jnp.int32),
        pltpu.VMEM((CH,), jnp.float32),
        pltpu.VMEM_SHARED((NDEG,), jnp.float32),
        pltpu.SemaphoreType.DMA,
    ],
)
def _deg_kernel(dstm, ones_hbm, zeros_hbm, out0, out1, idx_d, ones_v, acc,
                sem):
    cid = lax.axis_index("c")
    sid = lax.axis_index("s")
    wid = cid * 16 + sid
    pltpu.sync_copy(zeros_hbm, acc.at[pl.ds(sid * RPTD, RPTD)])
    pltpu.sync_copy(ones_hbm, ones_v)
    pltpu.sync_copy(dstm.at[pl.ds(wid * NCH, NCH)], idx_d)
    plsc.subcore_barrier()

    def body(i, carry):
        pltpu.async_copy(ones_v, acc.at[idx_d.at[i]], sem, add=True)
        return carry

    lax.fori_loop(0, NCH, body, 0)

    def drain(i, carry):
        pltpu.make_async_copy(ones_v, acc.at[idx_d.at[i]], sem).wait()
        return carry

    lax.fori_loop(0, NCH, drain, 0)
    plsc.subcore_barrier()

    @pl.when(cid == 0)
    def _():
        pltpu.sync_copy(acc.at[pl.ds(sid * RPTD, RPTD)],
                        out0.at[pl.ds(sid * RPTD, RPTD)])

    @pl.when(cid == 1)
    def _():
        pltpu.sync_copy(acc.at[pl.ds(sid * RPTD, RPTD)],
                        out1.at[pl.ds(sid * RPTD, RPTD)])


@functools.partial(
    pl.kernel,
    out_type=jax.ShapeDtypeStruct((2, NPAD, H), jnp.float32),
    mesh=_mesh,
    scratch_types=[
        pltpu.VMEM((NCH, CH), jnp.int32),
        pltpu.VMEM((CH,), jnp.int32),
        pltpu.VMEM((CH,), jnp.int32),
        pltpu.VMEM((CH, H), jnp.float32),
        pltpu.VMEM((CH, H), jnp.float32),
        pltpu.VMEM_SHARED((NPAD, H), jnp.float32),
        pltpu.SemaphoreType.DMA,
        pltpu.SemaphoreType.DMA,
        pltpu.SemaphoreType.DMA,
        pltpu.SemaphoreType.DMA,
    ],
)
def _agg_kernel(h1s, srcm, dst1, zeros_hbm, out, idx_s, idx_d0, idx_d1,
                rows0, rows1, acc, sem0, sem1, semi0, semi1):
    cid = lax.axis_index("c")
    sid = lax.axis_index("s")
    wid = cid * 16 + sid
    base = wid * TPT
    @pl.when(cid == 0)
    def _():
        pltpu.sync_copy(h1s.at[pl.ds(sid * RPT, RPT)],
                        acc.at[pl.ds(sid * RPT, RPT)])

    @pl.when(cid == 1)
    def _():
        pltpu.sync_copy(zeros_hbm, acc.at[pl.ds(sid * RPT, RPT)])
    pltpu.sync_copy(srcm.at[pl.ds(wid * NCH, NCH)], idx_s)
    plsc.subcore_barrier()

    pltpu.async_copy(h1s.at[idx_s.at[0]], rows0, sem0)
    pltpu.async_copy(dst1.at[pl.ds(base, CH)], idx_d0, semi0)

    def body(j, carry):
        i0 = 2 * j
        pltpu.async_copy(h1s.at[idx_s.at[i0 + 1]], rows1, sem1)
        pltpu.async_copy(dst1.at[pl.ds(base + (i0 + 1) * CH, CH)], idx_d1,
                         semi1)
        pltpu.make_async_copy(h1s.at[idx_s.at[i0]], rows0, sem0).wait()
        pltpu.make_async_copy(dst1.at[pl.ds(base, CH)], idx_d0, semi0).wait()
        pltpu.sync_copy(rows0, acc.at[idx_d0], add=True)

        @pl.when(i0 + 2 < NCH)
        def _():
            pltpu.async_copy(h1s.at[idx_s.at[i0 + 2]], rows0, sem0)
            pltpu.async_copy(dst1.at[pl.ds(base + (i0 + 2) * CH, CH)], idx_d0,
                             semi0)

        pltpu.make_async_copy(h1s.at[idx_s.at[i0 + 1]], rows1, sem1).wait()
        pltpu.make_async_copy(dst1.at[pl.ds(base, CH)], idx_d1, semi1).wait()
        pltpu.sync_copy(rows1, acc.at[idx_d1], add=True)
        return carry

    lax.fori_loop(0, NCH // 2, body, 0)
    plsc.subcore_barrier()
    pltpu.sync_copy(acc.at[pl.ds(sid * RPT, RPT)],
                    out.at[cid, pl.ds(sid * RPT, RPT)])


def _prep_body(x_ref, w0_ref, degb0_ref, degb1_ref, t_ref, wg1_ref, bg1_ref,
               wg2_ref, bg2_ref, h1s_ref, dinvb_ref, gate_ref):
    pid = pl.program_id(0)
    deg = degb0_ref[...] + degb1_ref[...] + 1.0
    rows = jax.lax.broadcasted_iota(jnp.int32, (BR, H), 0) + pid * BR
    dinv = jnp.where(rows < N, jax.lax.rsqrt(deg), 0.0)
    mm = jax.lax.dot_general(x_ref[...], w0_ref[...], (((1,), (1,)), ((), ())),
                             preferred_element_type=jnp.float32)
    h1s_ref[...] = jnp.where(rows < N, dinv * mm, 0.0)
    dinvb_ref[...] = dinv
    t = t_ref[0, 0]
    g = jnp.tanh(t * wg1_ref[...] + bg1_ref[...])
    gate_ref[...] = jax.nn.sigmoid(
        jax.lax.dot_general(g, wg2_ref[...], (((1,), (1,)), ((), ())),
                            preferred_element_type=jnp.float32) + bg2_ref[...])


_prep_call = pl.pallas_call(
    _prep_body,
    grid=(GRID,),
    in_specs=[
        pl.BlockSpec((BR, H), lambda i: (i, 0)),
        pl.BlockSpec((H, H), lambda i: (0, 0)),
        pl.BlockSpec((BR, H), lambda i: (i, 0)),
        pl.BlockSpec((BR, H), lambda i: (i, 0)),
        pl.BlockSpec((1, 1), lambda i: (0, 0)),
        pl.BlockSpec((1, H), lambda i: (0, 0)),
        pl.BlockSpec((1, H), lambda i: (0, 0)),
        pl.BlockSpec((H, H), lambda i: (0, 0)),
        pl.BlockSpec((1, H), lambda i: (0, 0)),
    ],
    out_specs=[
        pl.BlockSpec((BR, H), lambda i: (i, 0)),
        pl.BlockSpec((BR, H), lambda i: (i, 0)),
        pl.BlockSpec((1, H), lambda i: (0, 0)),
    ],
    out_shape=[
        jax.ShapeDtypeStruct((NPAD, H), jnp.float32),
        jax.ShapeDtypeStruct((NPAD, H), jnp.float32),
        jax.ShapeDtypeStruct((1, H), jnp.float32),
    ],
)


def _layer_body(agg_ref, dinvb_ref, gate_ref, b_ref, w_ref, out_ref):
    s = agg_ref[0] + agg_ref[1]
    h = gate_ref[...] * jnp.maximum(dinvb_ref[...] * s + b_ref[...], 0.0)
    out_ref[...] = dinvb_ref[...] * jax.lax.dot_general(
        h, w_ref[...], (((1,), (1,)), ((), ())),
        preferred_element_type=jnp.float32)


_layer_call = pl.pallas_call(
    _layer_body,
    grid=(GRID,),
    in_specs=[
        pl.BlockSpec((2, BR, H), lambda i: (0, i, 0)),
        pl.BlockSpec((BR, H), lambda i: (i, 0)),
        pl.BlockSpec((1, H), lambda i: (0, 0)),
        pl.BlockSpec((1, H), lambda i: (0, 0)),
        pl.BlockSpec((H, H), lambda i: (0, 0)),
    ],
    out_specs=pl.BlockSpec((BR, H), lambda i: (i, 0)),
    out_shape=jax.ShapeDtypeStruct((NPAD, H), jnp.float32),
)


def _final_body(agg_ref, dinvb_ref, gate_ref, b_ref, out_ref):
    s = agg_ref[0] + agg_ref[1]
    out_ref[...] = gate_ref[...] * jnp.maximum(
        dinvb_ref[...] * s + b_ref[...], 0.0)


BRF = 1000

_final_call = pl.pallas_call(
    _final_body,
    grid=(N // BRF,),
    in_specs=[
        pl.BlockSpec((2, BRF, H), lambda i: (0, i, 0)),
        pl.BlockSpec((BRF, H), lambda i: (i, 0)),
        pl.BlockSpec((1, H), lambda i: (0, 0)),
        pl.BlockSpec((1, H), lambda i: (0, 0)),
    ],
    out_specs=pl.BlockSpec((BRF, H), lambda i: (i, 0)),
    out_shape=jax.ShapeDtypeStruct((N, H), jnp.float32),
)


def kernel(x, edge_index, timestamp, W0, b0, W1, b1, W2, b2, Wg1, bg1, Wg2, bg2):
    src = edge_index[0]
    dst = edge_index[1]
    pad_idx = N + (jnp.arange(EP - E, dtype=jnp.int32) % (NPAD - N))
    srcm = jnp.concatenate([src, pad_idx]).reshape(NW * NCH, CH)
    dst1 = jnp.concatenate([dst, pad_idx])
    dstm = dst1.reshape(NW * NCH, CH)
    zeros_h = jnp.zeros((RPT, H), jnp.float32)
    zeros_1 = jnp.zeros((RPTD,), jnp.float32)
    ones_1 = jnp.ones((CH,), jnp.float32)
    t = timestamp.reshape(1, 1)
    wg1r = Wg1.reshape(1, H)
    bg1r = bg1.reshape(1, H)
    bg2r = bg2.reshape(1, H)
    b0r = b0.reshape(1, H)
    b1r = b1.reshape(1, H)
    b2r = b2.reshape(1, H)

    deg0, deg1 = _deg_kernel(dstm, ones_1, zeros_1)
    degb0 = jnp.broadcast_to(deg0[:NPAD, None], (NPAD, H))
    degb1 = jnp.broadcast_to(deg1[:NPAD, None], (NPAD, H))
    h1s0, dinvb, gate = _prep_call(x, W0, degb0, degb1, t, wg1r, bg1r,
                                   Wg2, bg2r)
    agg0 = _agg_kernel(h1s0, srcm, dst1, zeros_h)
    h1s1 = _layer_call(agg0, dinvb, gate, b0r, W1)
    agg1 = _agg_kernel(h1s1, srcm, dst1, zeros_h)
    h1s2 = _layer_call(agg1, dinvb, gate, b1r, W2)
    agg2 = _agg_kernel(h1s2, srcm, dst1, zeros_h)
    return _final_call(agg2, dinvb, gate, b2r)


# speedup vs baseline: 1.3321x; 1.0121x over previous
import functools

import jax
import jax.numpy as jnp
from jax import lax
from jax.experimental import pallas as pl
from jax.experimental.pallas import tpu as pltpu
from jax.experimental.pallas import tpu_sc as plsc

N = 10000
H = 128
E = 320000

NPAD = 10112
NW = 32
CH = 128
NCH = 80
TPT = NCH * CH
EP = NW * TPT
RPT = NPAD // 16
NDEG = 10240
RPTD = NDEG // 16
BR = 5056
GRID = NPAD // BR

_mesh = plsc.VectorSubcoreMesh(core_axis_name="c", subcore_axis_name="s",
                               num_cores=2, num_subcores=16)


@functools.partial(
    pl.kernel,
    out_type=[jax.ShapeDtypeStruct((NDEG,), jnp.float32),
              jax.ShapeDtypeStruct((NDEG,), jnp.float32)],
    mesh=_mesh,
    scratch_types=[
        pltpu.VMEM((NCH, CH), jnp.int32),
        pltpu.VMEM((CH,), jnp.float32),
        pltpu.VMEM_SHARED((NDEG,), jnp.float32),
        pltpu.SemaphoreType.DMA,
    ],
)
def _deg_kernel(dstm, ones_hbm, zeros_hbm, out0, out1, idx_d, ones_v, acc,
                sem):
    cid = lax.axis_index("c")
    sid = lax.axis_index("s")
    wid = cid * 16 + sid
    pltpu.sync_copy(zeros_hbm, acc.at[pl.ds(sid * RPTD, RPTD)])
    pltpu.sync_copy(ones_hbm, ones_v)
    pltpu.sync_copy(dstm.at[pl.ds(wid * NCH, NCH)], idx_d)
    plsc.subcore_barrier()

    def body(i, carry):
        pltpu.async_copy(ones_v, acc.at[idx_d.at[i]], sem, add=True)
        return carry

    lax.fori_loop(0, NCH, body, 0)

    def drain(i, carry):
        pltpu.make_async_copy(ones_v, acc.at[idx_d.at[i]], sem).wait()
        return carry

    lax.fori_loop(0, NCH, drain, 0)
    plsc.subcore_barrier()

    @pl.when(cid == 0)
    def _():
        pltpu.sync_copy(acc.at[pl.ds(sid * RPTD, RPTD)],
                        out0.at[pl.ds(sid * RPTD, RPTD)])

    @pl.when(cid == 1)
    def _():
        pltpu.sync_copy(acc.at[pl.ds(sid * RPTD, RPTD)],
                        out1.at[pl.ds(sid * RPTD, RPTD)])


@functools.partial(
    pl.kernel,
    out_type=jax.ShapeDtypeStruct((2, NPAD, H), jnp.float32),
    mesh=_mesh,
    scratch_types=[
        pltpu.VMEM((NCH, CH), jnp.int32),
        pltpu.VMEM((CH,), jnp.int32),
        pltpu.VMEM((CH,), jnp.int32),
        pltpu.VMEM((CH, H), jnp.float32),
        pltpu.VMEM((CH, H), jnp.float32),
        pltpu.VMEM_SHARED((NPAD, H), jnp.float32),
        pltpu.SemaphoreType.DMA,
        pltpu.SemaphoreType.DMA,
        pltpu.SemaphoreType.DMA,
        pltpu.SemaphoreType.DMA,
    ],
)
def _agg_kernel(h1s, srcm, dst1, zeros_hbm, out, idx_s, idx_d0, idx_d1,
                rows0, rows1, acc, sem0, sem1, semi0, semi1):
    cid = lax.axis_index("c")
    sid = lax.axis_index("s")
    wid = cid * 16 + sid
    base = wid * TPT
    @pl.when(cid == 0)
    def _():
        pltpu.sync_copy(h1s.at[pl.ds(sid * RPT, RPT)],
                        acc.at[pl.ds(sid * RPT, RPT)])

    @pl.when(cid == 1)
    def _():
        pltpu.sync_copy(zeros_hbm, acc.at[pl.ds(sid * RPT, RPT)])
    pltpu.sync_copy(srcm.at[pl.ds(wid * NCH, NCH)], idx_s)
    plsc.subcore_barrier()

    pltpu.async_copy(h1s.at[idx_s.at[0]], rows0, sem0)
    pltpu.async_copy(dst1.at[pl.ds(base, CH)], idx_d0, semi0)

    def body(j, carry):
        i0 = 2 * j
        pltpu.async_copy(h1s.at[idx_s.at[i0 + 1]], rows1, sem1)
        pltpu.async_copy(dst1.at[pl.ds(base + (i0 + 1) * CH, CH)], idx_d1,
                         semi1)
        pltpu.make_async_copy(h1s.at[idx_s.at[i0]], rows0, sem0).wait()
        pltpu.make_async_copy(dst1.at[pl.ds(base, CH)], idx_d0, semi0).wait()
        pltpu.sync_copy(rows0, acc.at[idx_d0], add=True)

        @pl.when(i0 + 2 < NCH)
        def _():
            pltpu.async_copy(h1s.at[idx_s.at[i0 + 2]], rows0, sem0)
            pltpu.async_copy(dst1.at[pl.ds(base + (i0 + 2) * CH, CH)], idx_d0,
                             semi0)

        pltpu.make_async_copy(h1s.at[idx_s.at[i0 + 1]], rows1, sem1).wait()
        pltpu.make_async_copy(dst1.at[pl.ds(base, CH)], idx_d1, semi1).wait()
        pltpu.sync_copy(rows1, acc.at[idx_d1], add=True)
        return carry

    lax.fori_loop(0, NCH // 2, body, 0)
    plsc.subcore_barrier()
    pltpu.sync_copy(acc.at[pl.ds(sid * RPT, RPT)],
                    out.at[cid, pl.ds(sid * RPT, RPT)])


def _prep_body(x_ref, w0_ref, degb0_ref, degb1_ref, t_ref, wg1_ref, bg1_ref,
               wg2_ref, bg2_ref, h1s_ref, dinvb_ref, gate_ref):
    pid = pl.program_id(0)
    deg = degb0_ref[...] + degb1_ref[...] + 1.0
    rows = jax.lax.broadcasted_iota(jnp.int32, (BR, H), 0) + pid * BR
    dinv = jnp.where(rows < N, jax.lax.rsqrt(deg), 0.0)
    mm = jax.lax.dot_general(x_ref[...], w0_ref[...], (((1,), (1,)), ((), ())),
                             preferred_element_type=jnp.float32)
    h1s_ref[...] = jnp.where(rows < N, dinv * mm, 0.0)
    dinvb_ref[...] = dinv
    t = t_ref[0, 0]
    g = jnp.tanh(t * wg1_ref[...] + bg1_ref[...])
    gate_ref[...] = jax.nn.sigmoid(
        jax.lax.dot_general(g, wg2_ref[...], (((1,), (1,)), ((), ())),
                            preferred_element_type=jnp.float32) + bg2_ref[...])


_prep_call = pl.pallas_call(
    _prep_body,
    grid=(GRID,),
    in_specs=[
        pl.BlockSpec((BR, H), lambda i: (i, 0)),
        pl.BlockSpec((H, H), lambda i: (0, 0)),
        pl.BlockSpec((BR, H), lambda i: (i, 0)),
        pl.BlockSpec((BR, H), lambda i: (i, 0)),
        pl.BlockSpec((1, 1), lambda i: (0, 0)),
        pl.BlockSpec((1, H), lambda i: (0, 0)),
        pl.BlockSpec((1, H), lambda i: (0, 0)),
        pl.BlockSpec((H, H), lambda i: (0, 0)),
        pl.BlockSpec((1, H), lambda i: (0, 0)),
    ],
    out_specs=[
        pl.BlockSpec((BR, H), lambda i: (i, 0)),
        pl.BlockSpec((BR, H), lambda i: (i, 0)),
        pl.BlockSpec((1, H), lambda i: (0, 0)),
    ],
    out_shape=[
        jax.ShapeDtypeStruct((NPAD, H), jnp.float32),
        jax.ShapeDtypeStruct((NPAD, H), jnp.float32),
        jax.ShapeDtypeStruct((1, H), jnp.float32),
    ],
)


def _layer_body(agg_ref, dinvb_ref, gate_ref, b_ref, w_ref, out_ref):
    s = agg_ref[0] + agg_ref[1]
    h = gate_ref[...] * jnp.maximum(dinvb_ref[...] * s + b_ref[...], 0.0)
    out_ref[...] = dinvb_ref[...] * jax.lax.dot_general(
        h, w_ref[...], (((1,), (1,)), ((), ())),
        preferred_element_type=jnp.float32)


_layer_call = pl.pallas_call(
    _layer_body,
    grid=(GRID,),
    in_specs=[
        pl.BlockSpec((2, BR, H), lambda i: (0, i, 0)),
        pl.BlockSpec((BR, H), lambda i: (i, 0)),
        pl.BlockSpec((1, H), lambda i: (0, 0)),
        pl.BlockSpec((1, H), lambda i: (0, 0)),
        pl.BlockSpec((H, H), lambda i: (0, 0)),
    ],
    out_specs=pl.BlockSpec((BR, H), lambda i: (i, 0)),
    out_shape=jax.ShapeDtypeStruct((NPAD, H), jnp.float32),
)


def _final_body(agg_ref, dinvb_ref, gate_ref, b_ref, out_ref):
    s = agg_ref[0] + agg_ref[1]
    out_ref[...] = gate_ref[...] * jnp.maximum(
        dinvb_ref[...] * s + b_ref[...], 0.0)


BRF = 2000

_final_call = pl.pallas_call(
    _final_body,
    grid=(N // BRF,),
    in_specs=[
        pl.BlockSpec((2, BRF, H), lambda i: (0, i, 0)),
        pl.BlockSpec((BRF, H), lambda i: (i, 0)),
        pl.BlockSpec((1, H), lambda i: (0, 0)),
        pl.BlockSpec((1, H), lambda i: (0, 0)),
    ],
    out_specs=pl.BlockSpec((BRF, H), lambda i: (i, 0)),
    out_shape=jax.ShapeDtypeStruct((N, H), jnp.float32),
)


def kernel(x, edge_index, timestamp, W0, b0, W1, b1, W2, b2, Wg1, bg1, Wg2, bg2):
    src = edge_index[0]
    dst = edge_index[1]
    pad_idx = N + (jnp.arange(EP - E, dtype=jnp.int32) % (NPAD - N))
    srcm = jnp.concatenate([src, pad_idx]).reshape(NW * NCH, CH)
    dst1 = jnp.concatenate([dst, pad_idx])
    dstm = dst1.reshape(NW * NCH, CH)
    zeros_h = jnp.zeros((RPT, H), jnp.float32)
    zeros_1 = jnp.zeros((RPTD,), jnp.float32)
    ones_1 = jnp.ones((CH,), jnp.float32)
    t = timestamp.reshape(1, 1)
    wg1r = Wg1.reshape(1, H)
    bg1r = bg1.reshape(1, H)
    bg2r = bg2.reshape(1, H)
    b0r = b0.reshape(1, H)
    b1r = b1.reshape(1, H)
    b2r = b2.reshape(1, H)

    deg0, deg1 = _deg_kernel(dstm, ones_1, zeros_1)
    degb0 = jnp.broadcast_to(deg0[:NPAD, None], (NPAD, H))
    degb1 = jnp.broadcast_to(deg1[:NPAD, None], (NPAD, H))
    h1s0, dinvb, gate = _prep_call(x, W0, degb0, degb1, t, wg1r, bg1r,
                                   Wg2, bg2r)
    agg0 = _agg_kernel(h1s0, srcm, dst1, zeros_h)
    h1s1 = _layer_call(agg0, dinvb, gate, b0r, W1)
    agg1 = _agg_kernel(h1s1, srcm, dst1, zeros_h)
    h1s2 = _layer_call(agg1, dinvb, gate, b1r, W2)
    agg2 = _agg_kernel(h1s2, srcm, dst1, zeros_h)
    return _final_call(agg2, dinvb, gate, b2r)
